# Initial kernel scaffold; baseline (speedup 1.0000x reference)
import functools

import jax
import jax.numpy as jnp
import numpy as np
from jax import lax
from jax.experimental import pallas as pl
from jax.experimental.pallas import tpu as pltpu
from jax.experimental.pallas import tpu_sc as plsc

Q, K, D, C, H, P = 4096, 1000000, 64, 128, 8, 16
L = 16
NC, NS = 2, 16
NW = NC * NS
QPW = Q // NW
NG = C // L


def _nn_body(keys_hbm, cidx_hbm, q_hbm, nnd_hbm, nni_hbm,
             qblk, cidxblk, rows, resd, resi, sem):
    wid = lax.axis_index("s") * NC + lax.axis_index("c")
    base = wid * QPW

    pltpu.sync_copy(q_hbm.at[pl.ds(base, QPW)], qblk)
    pltpu.sync_copy(cidx_hbm.at[pl.ds(base, QPW)], cidxblk)

    lane = lax.iota(jnp.int32, L)
    lane0 = lane == 0
    row_ids = [lane + g * L for g in range(NG)]

    def per_query(i, carry):
        pltpu.async_copy(keys_hbm.at[cidxblk.at[i]], rows, sem).wait()

        ivec = jnp.full((L,), i, jnp.int32)

        def dstep(t, accs):
            accs = list(accs)
            for u in range(4):
                d = 4 * t + u
                dvec = jnp.full((L,), d, jnp.int32)
                qd = plsc.load_gather(qblk, [ivec, dvec])
                for g in range(NG):
                    kv = plsc.load_gather(rows, [row_ids[g], dvec])
                    diff = kv - qd
                    accs[g] = accs[g] + diff * diff
            return tuple(accs)

        zeros = tuple(jnp.zeros((L,), jnp.float32) for _ in range(NG))
        dists = lax.fori_loop(0, D // 4, dstep, zeros)

        bestd = dists[0]
        besti = cidxblk[i, pl.ds(0, L)]
        for g in range(1, NG):
            cg = cidxblk[i, pl.ds(g * L, L)]
            m = dists[g] < bestd
            bestd = jnp.where(m, dists[g], bestd)
            besti = jnp.where(m, cg, besti)

        sk, sv = plsc.sort_key_val(bestd, besti)
        plsc.store_scatter(resd, [ivec], sk, mask=lane0)
        plsc.store_scatter(resi, [ivec], sv, mask=lane0)
        return carry

    lax.fori_loop(0, QPW, per_query, 0)

    pltpu.sync_copy(resd, nnd_hbm.at[pl.ds(base, QPW)])
    pltpu.sync_copy(resi, nni_hbm.at[pl.ds(base, QPW)])


@jax.jit
def _nn_call(keys, cidx, queries):
    mesh = plsc.VectorSubcoreMesh(core_axis_name="c", subcore_axis_name="s")
    f = pl.kernel(
        _nn_body,
        out_type=(
            jax.ShapeDtypeStruct((Q,), jnp.float32),
            jax.ShapeDtypeStruct((Q,), jnp.int32),
        ),
        mesh=mesh,
        scratch_types=[
            pltpu.VMEM((QPW, D), jnp.float32),
            pltpu.VMEM((QPW, C), jnp.int32),
            pltpu.VMEM((C, D), jnp.float32),
            pltpu.VMEM((QPW,), jnp.float32),
            pltpu.VMEM((QPW,), jnp.int32),
            pltpu.SemaphoreType.DMA,
        ],
    )
    return f(keys, cidx, queries)


_SEL = np.zeros((H * P, H), np.float32)
for _j in range(H * P):
    _SEL[_j, _j // P] = float(2 ** (_j % P))


def _codes_body(q_ref, w_ref, s_ref, out_ref):
    vals = jnp.dot(q_ref[...], w_ref[...], preferred_element_type=jnp.float32)
    bits = (vals > 0.0).astype(jnp.float32)
    codes = lax.dot_general(s_ref[...], bits, (((0,), (1,)), ((), ())),
                            preferred_element_type=jnp.float32)
    out_ref[...] = codes.astype(jnp.int32)


@jax.jit
def _codes_call(queries, projections):
    w = jnp.transpose(projections, (1, 0, 2)).reshape(D, H * P)
    return pl.pallas_call(
        _codes_body,
        out_shape=jax.ShapeDtypeStruct((H, Q), jnp.int32),
    )(queries, w, jnp.asarray(_SEL))


def kernel(queries, keys, projections, candidate_idx):
    codes = _codes_call(queries, projections)
    nn_dist, nn_idx = _nn_call(keys, candidate_idx.astype(jnp.int32), queries)
    return nn_dist, nn_idx, codes


# speedup vs baseline: 2.4682x; 2.4682x over previous
import jax
import jax.numpy as jnp
import numpy as np
from jax import lax
from jax.experimental import pallas as pl
from jax.experimental.pallas import tpu as pltpu
from jax.experimental.pallas import tpu_sc as plsc

Q, K, D, C, H, P = 4096, 1000000, 64, 128, 8, 16
L = 16
NC, NS = 2, 16
NW = NC * NS
QPW = Q // NW
NG = C // L
RB = 16384
SPLIT = 30 * RB
TBL = K - SPLIT
NTAIL = -(-(K - 2 * SPLIT) // RB)


def _nn_body(keys2_hbm, cidx_hbm, q_hbm, nnd_hbm, nni_hbm,
             qblk, cidxblk, rows_a, rows_b, rows_c, rows_d,
             idx_a, idx_b, idx_c, idx_d_buf,
             resd, resi, sem_a, sem_b, sem_c, sem_d):
    wid = lax.axis_index("s") * NC + lax.axis_index("c")
    base = wid * QPW

    pltpu.sync_copy(q_hbm.at[pl.ds(base, QPW)], qblk)
    pltpu.sync_copy(cidx_hbm.at[pl.ds(base, QPW)], cidxblk)

    lane = lax.iota(jnp.int32, L)
    lane0 = lane == 0
    row_ids = [lane + g * L for g in range(NG)]

    def compute_query(i, rows):
        ivec = jnp.full((L,), i, jnp.int32)

        cgs, offs = [], []
        for g in range(NG):
            cg = cidxblk[i, pl.ds(g * L, L)]
            cgs.append(cg)
            mid = (cg >= SPLIT) & (cg < 2 * SPLIT)
            offs.append(jnp.where(mid, D, 0).astype(jnp.int32))

        def dstep(t, accs):
            accs = list(accs)
            for u in range(4):
                d = 4 * t + u
                idx_d = (jnp.full((L,), d, jnp.int32) + lane) & (D - 1)
                qd = plsc.load_gather(qblk, [ivec, idx_d])
                for g in range(NG):
                    kv = plsc.load_gather(rows, [row_ids[g], offs[g] | idx_d])
                    diff = kv - qd
                    accs[g] = accs[g] + diff * diff
            return tuple(accs)

        zeros = tuple(jnp.zeros((L,), jnp.float32) for _ in range(NG))
        dists = lax.fori_loop(0, D // 4, dstep, zeros)

        bestd = dists[0]
        besti = cgs[0]
        for g in range(1, NG):
            m = dists[g] < bestd
            bestd = jnp.where(m, dists[g], bestd)
            besti = jnp.where(m, cgs[g], besti)

        sk, sv = plsc.sort_key_val(bestd, besti)
        plsc.store_scatter(resd, [ivec], sk, mask=lane0)
        plsc.store_scatter(resi, [ivec], sv, mask=lane0)

    def start_gather(i, rows, sem, idxb):
        for g in range(NG):
            cg = cidxblk[i, pl.ds(g * L, L)]
            idxb[pl.ds(g * L, L)] = jnp.where(cg >= SPLIT, cg - SPLIT, cg)
        pltpu.async_copy(keys2_hbm.at[idxb], rows, sem)

    def wait_gather(rows, sem, idxb):
        pltpu.make_async_copy(keys2_hbm.at[idxb], rows, sem).wait()

    bufs = [(rows_a, sem_a, idx_a), (rows_b, sem_b, idx_b),
            (rows_c, sem_c, idx_c), (rows_d, sem_d, idx_d_buf)]
    for j in range(3):
        start_gather(j, *bufs[j])

    def per_quad(t, carry):
        for j in range(4):
            i = 4 * t + j
            rows, sem, idxb = bufs[j]
            wait_gather(rows, sem, idxb)
            compute_query(i, rows)
            nxt = jnp.minimum(i + 3, QPW - 1)
            start_gather(nxt, *bufs[(j + 3) % 4])
        return carry

    lax.fori_loop(0, QPW // 4, per_quad, 0)
    for j in range(3):
        wait_gather(*bufs[j])

    pltpu.sync_copy(resd, nnd_hbm.at[pl.ds(base, QPW)])
    pltpu.sync_copy(resi, nni_hbm.at[pl.ds(base, QPW)])


def _tr_body(a_ref, b_ref, out_ref):
    out_ref[:, 0:D] = jnp.transpose(a_ref[...])
    out_ref[:, D:2 * D] = jnp.transpose(b_ref[...])


@jax.jit
def _pair_table(keys):
    kt = jnp.transpose(keys)
    nb = SPLIT // RB
    return pl.pallas_call(
        _tr_body,
        grid=(nb + NTAIL,),
        in_specs=[
            pl.BlockSpec((D, RB), lambda t: (0, jnp.where(t < nb, t, t + nb))),
            pl.BlockSpec((D, RB), lambda t: (0, jnp.where(t < nb, t + nb, t + nb))),
        ],
        out_specs=pl.BlockSpec((RB, 2 * D), lambda t: (t, 0)),
        out_shape=jax.ShapeDtypeStruct((TBL, 2 * D), jnp.float32),
        compiler_params=pltpu.CompilerParams(
            dimension_semantics=("arbitrary",)),
    )(kt, kt)


@jax.jit
def _nn_call(keys2, cidx, queries):
    mesh = plsc.VectorSubcoreMesh(core_axis_name="c", subcore_axis_name="s")
    f = pl.kernel(
        _nn_body,
        out_type=(
            jax.ShapeDtypeStruct((Q,), jnp.float32),
            jax.ShapeDtypeStruct((Q,), jnp.int32),
        ),
        mesh=mesh,
        scratch_types=[
            pltpu.VMEM((QPW, D), jnp.float32),
            pltpu.VMEM((QPW, C), jnp.int32),
            pltpu.VMEM((C, 2 * D), jnp.float32),
            pltpu.VMEM((C, 2 * D), jnp.float32),
            pltpu.VMEM((C, 2 * D), jnp.float32),
            pltpu.VMEM((C, 2 * D), jnp.float32),
            pltpu.VMEM((C,), jnp.int32),
            pltpu.VMEM((C,), jnp.int32),
            pltpu.VMEM((C,), jnp.int32),
            pltpu.VMEM((C,), jnp.int32),
            pltpu.VMEM((QPW,), jnp.float32),
            pltpu.VMEM((QPW,), jnp.int32),
            pltpu.SemaphoreType.DMA,
            pltpu.SemaphoreType.DMA,
            pltpu.SemaphoreType.DMA,
            pltpu.SemaphoreType.DMA,
        ],
        compiler_params=pltpu.CompilerParams(
            needs_layout_passes=False, use_tc_tiling_on_sc=False),
    )
    return f(keys2, cidx, queries)


_SEL = np.zeros((H * P, H), np.float32)
for _j in range(H * P):
    _SEL[_j, _j // P] = float(2 ** (_j % P))


def _codes_body(q_ref, w_ref, s_ref, out_ref):
    vals = jnp.dot(q_ref[...], w_ref[...], preferred_element_type=jnp.float32)
    bits = (vals > 0.0).astype(jnp.float32)
    codes = lax.dot_general(s_ref[...], bits, (((0,), (1,)), ((), ())),
                            preferred_element_type=jnp.float32)
    out_ref[...] = codes.astype(jnp.int32)


@jax.jit
def _codes_call(queries, projections):
    w = jnp.transpose(projections, (1, 0, 2)).reshape(D, H * P)
    return pl.pallas_call(
        _codes_body,
        out_shape=jax.ShapeDtypeStruct((H, Q), jnp.int32),
    )(queries, w, jnp.asarray(_SEL))


def kernel(queries, keys, projections, candidate_idx):
    codes = _codes_call(queries, projections)
    keys2 = _pair_table(keys)
    nn_dist, nn_idx = _nn_call(keys2, candidate_idx.astype(jnp.int32), queries)
    return nn_dist, nn_idx, codes
